# baseline (speedup 1.0000x reference)
import functools

import jax
import jax.numpy as jnp
from jax import lax
from jax.experimental import pallas as pl
from jax.experimental.pallas import tpu as pltpu
from jax.experimental.pallas import tpu_sc as plsc

_info = plsc.get_sparse_core_info()
_NC, _NS = _info.num_cores, _info.num_subcores
_NW = _NC * _NS

_IDXW = 128
_CHUNK = 512
_GPC = _CHUNK // _IDXW


@functools.lru_cache(maxsize=None)
def _make_emb(n_rows, emb_dim):
    assert n_rows % (_NW * _CHUNK) == 0
    b_per_w = n_rows // _NW
    n_chunk = b_per_w // _CHUNK
    idx_rows_per_w = b_per_w // _IDXW

    mesh = plsc.VectorSubcoreMesh(core_axis_name="c", subcore_axis_name="s")

    @functools.partial(
        pl.kernel,
        mesh=mesh,
        compiler_params=pltpu.CompilerParams(use_tc_tiling_on_sc=False),
        out_type=jax.ShapeDtypeStruct((n_rows, emb_dim), jnp.float32),
        scratch_types=[
            pltpu.VMEM((idx_rows_per_w, _IDXW), jnp.int32),
            pltpu.VMEM((_CHUNK, emb_dim), jnp.float32),
            pltpu.VMEM((_CHUNK, emb_dim), jnp.float32),
            pltpu.SemaphoreType.DMA,
            pltpu.SemaphoreType.DMA,
        ],
    )
    def emb(table_hbm, idx_hbm, out_hbm, idx_v, rows_a, rows_b, sem_g, sem_s):
        wid = lax.axis_index("s") * _NC + lax.axis_index("c")
        base = wid * b_per_w

        pltpu.sync_copy(idx_hbm.at[pl.ds(wid * idx_rows_per_w, idx_rows_per_w)],
                        idx_v)

        bufs = (rows_a, rows_b)

        def start_gather(c):
            buf = bufs[c % 2]
            cps = []
            for k in range(_GPC):
                cps.append(pltpu.async_copy(
                    table_hbm.at[idx_v.at[c * _GPC + k]],
                    buf.at[pl.ds(k * _IDXW, _IDXW)],
                    sem_g))
            return cps

        g_cp = [None] * n_chunk
        s_cp = [None] * n_chunk
        g_cp[0] = start_gather(0)
        for c in range(n_chunk):
            for cp in g_cp[c]:
                cp.wait()
            s_cp[c] = pltpu.async_copy(
                bufs[c % 2], out_hbm.at[pl.ds(base + c * _CHUNK, _CHUNK)],
                sem_s)
            if c + 1 < n_chunk:
                if c >= 1:
                    s_cp[c - 1].wait()
                g_cp[c + 1] = start_gather(c + 1)
        if n_chunk >= 2:
            s_cp[n_chunk - 2].wait()
        s_cp[n_chunk - 1].wait()

    return emb


def kernel(idx, table):
    shape = idx.shape
    flat = idx.reshape(-1).astype(jnp.int32)
    emb = _make_emb(flat.shape[0], table.shape[1])
    out = emb(table, flat.reshape(-1, _IDXW))
    return out.reshape(*shape, table.shape[1])


# speedup vs baseline: 1.7425x; 1.7425x over previous
import functools

import jax
import jax.numpy as jnp
from jax import lax
from jax.experimental import pallas as pl
from jax.experimental.pallas import tpu as pltpu
from jax.experimental.pallas import tpu_sc as plsc

_info = plsc.get_sparse_core_info()
_NC, _NS = _info.num_cores, _info.num_subcores
_NW = _NC * _NS

_CHUNK = 256
_GROUPS = _CHUNK // 16


@functools.lru_cache(maxsize=None)
def _make_emb(n_rows, emb_dim):
    assert n_rows % (_NW * _CHUNK) == 0
    bpw = n_rows // _NW
    n_chunk = bpw // _CHUNK
    idx_rows = bpw // 128

    mesh = plsc.VectorSubcoreMesh(core_axis_name="c", subcore_axis_name="s")

    @functools.partial(
        pl.kernel,
        mesh=mesh,
        compiler_params=pltpu.CompilerParams(use_tc_tiling_on_sc=True),
        out_type=jax.ShapeDtypeStruct((n_rows, emb_dim), jnp.float32),
        scratch_types=[
            pltpu.VMEM((idx_rows, 128), jnp.int32),
            pltpu.VMEM((_CHUNK, emb_dim), jnp.float32),
            pltpu.VMEM((_CHUNK, emb_dim), jnp.float32),
            pltpu.SemaphoreType.DMA,
            pltpu.SemaphoreType.DMA,
            pltpu.SemaphoreType.DMA,
            pltpu.SemaphoreType.DMA,
        ],
    )
    def emb(table, idx2d, out, idxv, buf_a, buf_b, sga, sgb, ssa, ssb):
        wid = lax.axis_index("s") * _NC + lax.axis_index("c")
        base = wid * bpw
        pltpu.sync_copy(idx2d.at[pl.ds(wid * idx_rows, idx_rows)], idxv)

        bufs = (buf_a, buf_b)
        gsems = (sga, sgb)
        ssems = (ssa, ssb)

        def issue_fill(c, buf, gsem):
            def body(g, carry):
                flat = c * _CHUNK + g * 16
                vec = idxv.at[flat // 128][pl.ds(lax.rem(flat, 128), 16)]
                for j in range(16):
                    r = vec[j]
                    pltpu.async_copy(
                        table.at[pl.ds(r, 1)],
                        buf.at[pl.ds(g * 16 + j, 1)], gsem)
                return carry
            lax.fori_loop(0, _GROUPS, body, 0)

        def drain_fill(buf, gsem):
            pltpu.make_async_copy(table.at[pl.ds(0, _CHUNK)], buf, gsem).wait()

        def drain_store(buf, ssem):
            pltpu.make_async_copy(out.at[pl.ds(0, _CHUNK)], buf, ssem).wait()

        issue_fill(0, buf_a, sga)

        def loop_body(c, carry):
            for par in range(2):
                @pl.when(lax.rem(c, 2) == par)
                def _():
                    buf, gsem, ssem = bufs[par], gsems[par], ssems[par]
                    obuf, ogsem, ossem = (bufs[1 - par], gsems[1 - par],
                                          ssems[1 - par])
                    @pl.when(c + 1 < n_chunk)
                    def _():
                        @pl.when(c >= 1)
                        def _():
                            drain_store(obuf, ossem)
                        issue_fill(c + 1, obuf, ogsem)
                    drain_fill(buf, gsem)
                    pltpu.async_copy(
                        buf, out.at[pl.ds(base + c * _CHUNK, _CHUNK)], ssem)
            return carry
        lax.fori_loop(0, n_chunk, loop_body, 0)
        drain_store(bufs[(n_chunk - 2) % 2], ssems[(n_chunk - 2) % 2])
        drain_store(bufs[(n_chunk - 1) % 2], ssems[(n_chunk - 1) % 2])

    return emb


def kernel(idx, table):
    shape = idx.shape
    flat = idx.reshape(-1).astype(jnp.int32)
    emb = _make_emb(flat.shape[0], table.shape[1])
    out = emb(table, flat.reshape(-1, 128))
    return out.reshape(*shape, table.shape[1])


# speedup vs baseline: 3.5263x; 2.0237x over previous
import functools

import jax
import jax.numpy as jnp
from jax import lax
from jax.experimental import pallas as pl
from jax.experimental.pallas import tpu as pltpu
from jax.experimental.pallas import tpu_sc as plsc

_info = plsc.get_sparse_core_info()
_NC, _NS = _info.num_cores, _info.num_subcores

_NENT = 1000000
_D = 64
_DPC = _D // _NC
_NL = 10
_B = 16384
_IW = _B // _NS

mesh = plsc.VectorSubcoreMesh(core_axis_name="c", subcore_axis_name="s")


@functools.partial(
    pl.kernel,
    mesh=mesh,
    compiler_params=pltpu.CompilerParams(use_tc_tiling_on_sc=True),
    out_type=jax.ShapeDtypeStruct((_NL * _D, _B), jnp.float32),
    scratch_types=[
        pltpu.VMEM_SHARED((_NENT,), jnp.float32),
        pltpu.VMEM_SHARED((_NENT,), jnp.float32),
        pltpu.VMEM((_IW,), jnp.int32),
        pltpu.VMEM((_IW,), jnp.int32),
        pltpu.VMEM((_IW,), jnp.float32),
        pltpu.VMEM((_IW,), jnp.float32),
        pltpu.SemaphoreType.DMA,
        pltpu.SemaphoreType.DMA,
        pltpu.SemaphoreType.DMA,
        pltpu.SemaphoreType.DMA,
        pltpu.SemaphoreType.DMA,
        pltpu.SemaphoreType.DMA,
        pltpu.SemaphoreType.DMA,
    ],
)
def _emb(t64, idx10, out, spm_a, spm_b, il_a, il_b, gb_a, gb_b,
         sla, slb, sia, sib, sg, soa, sob):
    cid = lax.axis_index("c")
    sid = lax.axis_index("s")
    dbase = cid * _DPC
    ibase = sid * _IW

    spms = (spm_a, spm_b)
    lsems = (sla, slb)
    ils = (il_a, il_b)
    isems = (sia, sib)
    gbs = (gb_a, gb_b)
    osems = (soa, sob)

    def start_load(d, buf, sem):
        @pl.when(sid == 0)
        def _():
            pltpu.async_copy(t64.at[d], buf, sem)

    def drain_load(buf, sem):
        @pl.when(sid == 0)
        def _():
            pltpu.make_async_copy(t64.at[0], buf, sem).wait()

    def issue_il(ft, p):
        pltpu.async_copy(idx10.at[ft].at[pl.ds(ibase, _IW)], ils[p], isems[p])

    def drain_il(p):
        pltpu.make_async_copy(idx10.at[0].at[pl.ds(0, _IW)], ils[p],
                              isems[p]).wait()

    def drain_g(p):
        pltpu.make_async_copy(out.at[0].at[pl.ds(0, _IW)], gbs[p], sg).wait()

    def issue_out(d, ft, p):
        pltpu.async_copy(gbs[p], out.at[ft * _D + d].at[pl.ds(ibase, _IW)],
                         osems[p])

    def drain_out(p):
        pltpu.make_async_copy(out.at[0].at[pl.ds(0, _IW)], gbs[p],
                              osems[p]).wait()

    start_load(dbase, spm_a, sla)
    issue_il(0, 0)

    def d_body(dloc, carry):
        d = dbase + dloc
        for par in range(2):
            @pl.when(lax.rem(dloc, 2) == par)
            def _():
                spm, lsem = spms[par], lsems[par]
                drain_load(spm, lsem)
                plsc.subcore_barrier()
                @pl.when(dloc + 1 < _DPC)
                def _():
                    start_load(d + 1, spms[1 - par], lsems[1 - par])
                for ft in range(_NL):
                    p = ft % 2
                    if ft >= 2:
                        drain_out(p)
                    else:
                        @pl.when(dloc >= 1)
                        def _():
                            drain_out(p)
                    drain_il(p)
                    pltpu.async_copy(spm.at[ils[p]], gbs[p], sg)
                    if ft == 0:
                        issue_il(1, 1)
                    else:
                        drain_g(1 - p)
                        issue_out(d, ft - 1, 1 - p)
                        issue_il(ft + 1 if ft < _NL - 1 else 0, 1 - p)
                drain_g(1)
                issue_out(d, _NL - 1, 1)
                plsc.subcore_barrier()
        return carry

    lax.fori_loop(0, _DPC, d_body, 0)
    drain_il(0)
    drain_out(0)
    drain_out(1)


def kernel(idx, table):
    idx10 = idx.transpose(1, 2, 0).reshape(_NL, _B).astype(jnp.int32)
    out2 = _emb(table.T, idx10)
    return out2.reshape(5, 2, _D, _B).transpose(3, 0, 1, 2)
